# Initial kernel scaffold; baseline (speedup 1.0000x reference)
import jax
import jax.numpy as jnp
from jax.experimental import pallas as pl


def kernel(node_feats, attr_feats, edge_index, Wn0, bn0, Wn1, bn1, Wa0, ba0, Wa1, ba1, edge_attention):
    raise NotImplementedError("write your pallas kernel here")


# speedup vs baseline: 2.1888x; 2.1888x over previous
import functools

import jax
import jax.numpy as jnp
from jax import lax
from jax.experimental import pallas as pl
from jax.experimental.pallas import tpu as pltpu
from jax.experimental.pallas import tpu_sc as plsc

_N = 10000
_D = 256
_E = 160000
_NP = 10240
_HALF = _NP // 2
_STRIPE = _HALF // 16
_W = 80
_TPW = _E // 16
_NWIN = _TPW // _W
_BLK = 128
_NG = _NP // _BLK

_mesh = plsc.VectorSubcoreMesh(core_axis_name="c", subcore_axis_name="s")


@functools.partial(
    pl.kernel,
    out_type=jax.ShapeDtypeStruct((_NP,), jnp.float32),
    mesh=_mesh,
    scratch_types=[
        pltpu.VMEM_SHARED((_NP,), jnp.float32),
        pltpu.VMEM((_W,), jnp.int32),
        pltpu.VMEM((_W,), jnp.float32),
        pltpu.VMEM((_NP // 16,), jnp.float32),
        pltpu.SemaphoreType.DMA,
    ],
)
def _sc_hist(gidx, zeros1d, out, acc, idx_g, ones_v, zbuf, sem):
    c = lax.axis_index("c")
    t = lax.axis_index("s")
    seg = _NP // 16
    pltpu.sync_copy(zeros1d, zbuf)
    pltpu.sync_copy(zbuf, acc.at[pl.ds(t * seg, seg)])
    for j in range(_W // 16):
        ones_v[pl.ds(j * 16, 16)] = jnp.full((16,), 1.0, jnp.float32)
    plsc.subcore_barrier()

    ebase = t * _TPW

    def w_body(w, carry):
        eoff = pl.multiple_of(ebase + w * _W, 8)
        pltpu.sync_copy(gidx.at[pl.ds(eoff, _W)], idx_g)
        pltpu.sync_copy(ones_v, acc.at[idx_g], add=True)
        return carry

    lax.fori_loop(0, _NWIN, w_body, 0)
    plsc.subcore_barrier()
    off = c * _HALF + t * _STRIPE
    pltpu.sync_copy(acc.at[pl.ds(off, _STRIPE)], zbuf.at[pl.ds(0, _STRIPE)])
    pltpu.sync_copy(zbuf.at[pl.ds(0, _STRIPE)], out.at[pl.ds(off, _STRIPE)])


_HD = _D // 2


@functools.partial(
    pl.kernel,
    out_type=(jax.ShapeDtypeStruct((_NP, _HD), jnp.float32),
              jax.ShapeDtypeStruct((_NP, _HD), jnp.float32)),
    mesh=_mesh,
    scratch_types=[
        pltpu.VMEM_SHARED((_HALF + 8, _HD), jnp.float32),
        pltpu.VMEM_SHARED((_HALF + 8, _HD), jnp.float32),
        pltpu.VMEM((_W,), jnp.int32),
        pltpu.VMEM((_W,), jnp.int32),
        pltpu.VMEM((_W, _HD), jnp.float32),
        pltpu.VMEM((_W, _HD), jnp.float32),
        pltpu.SemaphoreType.DMA,
    ],
)
def _sc_pass(tA, tB, gidx, sidx, zeros2d, outA, outB,
             accA, accB, idx_g, idx_l, rA, rB, sem):
    c = lax.axis_index("c")
    t = lax.axis_index("s")
    base = c * _HALF
    pltpu.sync_copy(zeros2d, rA)
    for k in range(_STRIPE // _W):
        pltpu.sync_copy(rA, accA.at[pl.ds(t * _STRIPE + k * _W, _W)])
        pltpu.sync_copy(rA, accB.at[pl.ds(t * _STRIPE + k * _W, _W)])

    @pl.when(t == 0)
    def _():
        pltpu.sync_copy(rA.at[pl.ds(0, 8)], accA.at[pl.ds(_HALF, 8)])
        pltpu.sync_copy(rA.at[pl.ds(0, 8)], accB.at[pl.ds(_HALF, 8)])

    plsc.subcore_barrier()

    ebase = t * _TPW

    def w_body(w, carry):
        eoff = pl.multiple_of(ebase + w * _W, 8)
        pltpu.sync_copy(gidx.at[pl.ds(eoff, _W)], idx_g)
        pltpu.sync_copy(sidx.at[pl.ds(eoff, _W)], idx_l)
        cpA = pltpu.async_copy(tA.at[idx_g], rA, sem)
        cpB = pltpu.async_copy(tB.at[idx_g], rB, sem)
        for j in range(_W // 16):
            d = idx_l[pl.ds(j * 16, 16)]
            ld = d - base
            ok = (ld >= 0) & (ld < _HALF)
            idx_l[pl.ds(j * 16, 16)] = jnp.where(ok, ld, _HALF)
        cpA.wait()
        cpB.wait()
        pltpu.sync_copy(rA, accA.at[idx_l], add=True)
        pltpu.sync_copy(rB, accB.at[idx_l], add=True)
        return carry

    lax.fori_loop(0, _NWIN, w_body, 0)
    plsc.subcore_barrier()
    for k in range(_STRIPE // _W):
        off = t * _STRIPE + k * _W
        pltpu.sync_copy(accA.at[pl.ds(off, _W)], rA)
        pltpu.sync_copy(rA, outA.at[pl.ds(base + off, _W)])
        pltpu.sync_copy(accB.at[pl.ds(off, _W)], rB)
        pltpu.sync_copy(rB, outB.at[pl.ds(base + off, _W)])


def _mm_score_body(x_ref, w_ref, b_ref, ea_ref, nm_ref, s_ref):
    z = jnp.dot(x_ref[...], w_ref[...], preferred_element_type=jnp.float32)
    nm = jnp.maximum(z + b_ref[...], 0.0)
    nm_ref[...] = nm
    s_ref[...] = jnp.sum(nm * ea_ref[...], axis=1)[None, None, :]


_mm_score = pl.pallas_call(
    _mm_score_body,
    grid=(_NG,),
    in_specs=[
        pl.BlockSpec((_BLK, _D), lambda i: (i, 0)),
        pl.BlockSpec((_D, _D), lambda i: (0, 0)),
        pl.BlockSpec((1, _D), lambda i: (0, 0)),
        pl.BlockSpec((1, _D), lambda i: (0, 0)),
    ],
    out_specs=[
        pl.BlockSpec((_BLK, _D), lambda i: (i, 0)),
        pl.BlockSpec((1, 1, _BLK), lambda i: (i, 0, 0)),
    ],
    out_shape=[
        jax.ShapeDtypeStruct((_NP, _D), jnp.float32),
        jax.ShapeDtypeStruct((_NG, 1, _BLK), jnp.float32),
    ],
)


def _mm_relu_body(xa_ref, xb_ref, wa_ref, wb_ref, b_ref, oa_ref, ob_ref):
    z = jnp.dot(xa_ref[...], wa_ref[...], preferred_element_type=jnp.float32)
    z += jnp.dot(xb_ref[...], wb_ref[...], preferred_element_type=jnp.float32)
    z = jnp.maximum(z + b_ref[...], 0.0)
    oa_ref[...] = z[:, :_HD]
    ob_ref[...] = z[:, _HD:]


_mm_relu = pl.pallas_call(
    _mm_relu_body,
    grid=(_NG,),
    in_specs=[
        pl.BlockSpec((_BLK, _HD), lambda i: (i, 0)),
        pl.BlockSpec((_BLK, _HD), lambda i: (i, 0)),
        pl.BlockSpec((_HD, _D), lambda i: (0, 0)),
        pl.BlockSpec((_HD, _D), lambda i: (0, 0)),
        pl.BlockSpec((1, _D), lambda i: (0, 0)),
    ],
    out_specs=[
        pl.BlockSpec((_BLK, _HD), lambda i: (i, 0)),
        pl.BlockSpec((_BLK, _HD), lambda i: (i, 0)),
    ],
    out_shape=[
        jax.ShapeDtypeStruct((_NP, _HD), jnp.float32),
        jax.ShapeDtypeStruct((_NP, _HD), jnp.float32),
    ],
)


def _softmax_body(s_ref, c_ref, p_ref):
    s = s_ref[...]
    flat = (lax.broadcasted_iota(jnp.int32, s.shape, 0) * _BLK
            + lax.broadcasted_iota(jnp.int32, s.shape, 1))
    valid = flat < _N
    m = jnp.max(jnp.where(valid, s, -jnp.inf))
    texp = jnp.where(valid, jnp.exp(s - m), 0.0)
    z = jnp.sum(c_ref[...] * texp)
    p_ref[...] = texp / z


_softmax = pl.pallas_call(
    _softmax_body,
    out_shape=jax.ShapeDtypeStruct((_NG, _BLK), jnp.float32),
)


def _scale_body(nm_ref, p_ref, qa_ref, qb_ref):
    q = nm_ref[...] * p_ref[...]
    qa_ref[...] = q[:, :_HD]
    qb_ref[...] = q[:, _HD:]


_scale = pl.pallas_call(
    _scale_body,
    grid=(_NG,),
    in_specs=[
        pl.BlockSpec((_BLK, _D), lambda i: (i, 0)),
        pl.BlockSpec((_BLK, 1), lambda i: (i, 0)),
    ],
    out_specs=[
        pl.BlockSpec((_BLK, _HD), lambda i: (i, 0)),
        pl.BlockSpec((_BLK, _HD), lambda i: (i, 0)),
    ],
    out_shape=[
        jax.ShapeDtypeStruct((_NP, _HD), jnp.float32),
        jax.ShapeDtypeStruct((_NP, _HD), jnp.float32),
    ],
)


def _addscale_body(h_ref, ua_ref, ub_ref, p_ref, o_ref):
    u = jnp.concatenate([ua_ref[...], ub_ref[...]], axis=1)
    o_ref[...] = h_ref[...] + u * p_ref[...]


_addscale = pl.pallas_call(
    _addscale_body,
    grid=(_NG,),
    in_specs=[
        pl.BlockSpec((_BLK, _D), lambda i: (i, 0)),
        pl.BlockSpec((_BLK, _HD), lambda i: (i, 0)),
        pl.BlockSpec((_BLK, _HD), lambda i: (i, 0)),
        pl.BlockSpec((_BLK, 1), lambda i: (i, 0)),
    ],
    out_specs=pl.BlockSpec((_BLK, _D), lambda i: (i, 0)),
    out_shape=jax.ShapeDtypeStruct((_NP, _D), jnp.float32),
)


def kernel(node_feats, attr_feats, edge_index, Wn0, bn0, Wn1, bn1,
           Wa0, ba0, Wa1, ba1, edge_attention):
    src = edge_index[0]
    dst = edge_index[1]
    h = jnp.pad(node_feats, ((0, _NP - _N), (0, 0)))
    ea = edge_attention.reshape(1, _D)
    zeros2d = jnp.zeros((_W, _HD), jnp.float32)
    zeros1d = jnp.zeros((_NP // 16,), jnp.float32)

    c_src = _sc_hist(src, zeros1d).reshape(_NG, _BLK)

    haA = haB = None
    for (Wn, bn, Wa, ba) in ((Wn0, bn0, Wa0, ba0), (Wn1, bn1, Wa1, ba1)):
        nm, s = _mm_score(h, Wn, bn.reshape(1, _D), ea)
        p = _softmax(s.reshape(_NG, _BLK), c_src)
        pcol = p.reshape(_NP, 1)
        qA, qB = _scale(nm, pcol)
        amA, amB = _sc_pass(qA, qB, src, dst, zeros2d)
        haA, haB = _mm_relu(amA, amB, Wa[:_HD], Wa[_HD:], ba.reshape(1, _D))
        uA, uB = _sc_pass(haA, haB, dst, src, zeros2d)
        h = _addscale(h, uA, uB, pcol)
    ha = jnp.concatenate([haA, haB], axis=1)
    return h[:_N], ha[:_N]


# speedup vs baseline: 2.2781x; 1.0408x over previous
import functools

import jax
import jax.numpy as jnp
from jax import lax
from jax.experimental import pallas as pl
from jax.experimental.pallas import tpu as pltpu
from jax.experimental.pallas import tpu_sc as plsc

_N = 10000
_D = 256
_E = 160000
_NP = 10240
_HALF = _NP // 2
_STRIPE = _HALF // 16
_W = 80
_TPW = _E // 16
_NWIN = _TPW // _W
_BLK = 128
_NG = _NP // _BLK

_mesh = plsc.VectorSubcoreMesh(core_axis_name="c", subcore_axis_name="s")


@functools.partial(
    pl.kernel,
    out_type=jax.ShapeDtypeStruct((_NP,), jnp.float32),
    mesh=_mesh,
    scratch_types=[
        pltpu.VMEM_SHARED((_NP,), jnp.float32),
        pltpu.VMEM((_W,), jnp.int32),
        pltpu.VMEM((_W,), jnp.float32),
        pltpu.VMEM((_NP // 16,), jnp.float32),
        pltpu.SemaphoreType.DMA,
    ],
)
def _sc_hist(gidx, zeros1d, out, acc, idx_g, ones_v, zbuf, sem):
    c = lax.axis_index("c")
    t = lax.axis_index("s")
    seg = _NP // 16
    pltpu.sync_copy(zeros1d, zbuf)
    pltpu.sync_copy(zbuf, acc.at[pl.ds(t * seg, seg)])
    for j in range(_W // 16):
        ones_v[pl.ds(j * 16, 16)] = jnp.full((16,), 1.0, jnp.float32)
    plsc.subcore_barrier()

    ebase = t * _TPW

    def w_body(w, carry):
        eoff = pl.multiple_of(ebase + w * _W, 8)
        pltpu.sync_copy(gidx.at[pl.ds(eoff, _W)], idx_g)
        pltpu.sync_copy(ones_v, acc.at[idx_g], add=True)
        return carry

    lax.fori_loop(0, _NWIN, w_body, 0)
    plsc.subcore_barrier()
    off = c * _HALF + t * _STRIPE
    pltpu.sync_copy(acc.at[pl.ds(off, _STRIPE)], zbuf.at[pl.ds(0, _STRIPE)])
    pltpu.sync_copy(zbuf.at[pl.ds(0, _STRIPE)], out.at[pl.ds(off, _STRIPE)])


_HD = _D // 2


@functools.partial(
    pl.kernel,
    out_type=(jax.ShapeDtypeStruct((_NP, _HD), jnp.float32),
              jax.ShapeDtypeStruct((_NP, _HD), jnp.float32)),
    mesh=_mesh,
    scratch_types=[
        pltpu.VMEM_SHARED((_HALF + 8, _HD), jnp.float32),
        pltpu.VMEM_SHARED((_HALF + 8, _HD), jnp.float32),
        pltpu.VMEM((_W,), jnp.int32),
        pltpu.VMEM((_W,), jnp.int32),
        pltpu.VMEM((_W, _HD), jnp.float32),
        pltpu.VMEM((_W, _HD), jnp.float32),
        pltpu.SemaphoreType.DMA,
    ],
)
def _sc_pass(tA, tB, gidx, sidx, zeros2d, outA, outB,
             accA, accB, idx_g, idx_l, rA, rB, sem):
    c = lax.axis_index("c")
    t = lax.axis_index("s")
    base = c * _HALF
    pltpu.sync_copy(zeros2d, rA)
    for k in range(_STRIPE // _W):
        pltpu.sync_copy(rA, accA.at[pl.ds(t * _STRIPE + k * _W, _W)])
        pltpu.sync_copy(rA, accB.at[pl.ds(t * _STRIPE + k * _W, _W)])

    @pl.when(t == 0)
    def _():
        pltpu.sync_copy(rA.at[pl.ds(0, 8)], accA.at[pl.ds(_HALF, 8)])
        pltpu.sync_copy(rA.at[pl.ds(0, 8)], accB.at[pl.ds(_HALF, 8)])

    plsc.subcore_barrier()

    ebase = t * _TPW

    def w_body(w, carry):
        eoff = pl.multiple_of(ebase + w * _W, 8)
        pltpu.sync_copy(gidx.at[pl.ds(eoff, _W)], idx_g)
        pltpu.sync_copy(sidx.at[pl.ds(eoff, _W)], idx_l)
        for j in range(_W // 16):
            d = idx_l[pl.ds(j * 16, 16)]
            g = idx_g[pl.ds(j * 16, 16)]
            ld = d - base
            ok = (ld >= 0) & (ld < _HALF)
            idx_l[pl.ds(j * 16, 16)] = jnp.where(ok, ld, -1)
            idx_g[pl.ds(j * 16, 16)] = jnp.where(ok, g, -1)
        gi = plsc.Indices(idx_g, ignored_value=-1)
        cpA = pltpu.async_copy(tA.at[gi], rA, sem)
        cpB = pltpu.async_copy(tB.at[gi], rB, sem)
        cpA.wait()
        cpB.wait()
        si = plsc.Indices(idx_l, ignored_value=-1)
        pltpu.sync_copy(rA, accA.at[si], add=True)
        pltpu.sync_copy(rB, accB.at[si], add=True)
        return carry

    lax.fori_loop(0, _NWIN, w_body, 0)
    plsc.subcore_barrier()
    for k in range(_STRIPE // _W):
        off = t * _STRIPE + k * _W
        pltpu.sync_copy(accA.at[pl.ds(off, _W)], rA)
        pltpu.sync_copy(rA, outA.at[pl.ds(base + off, _W)])
        pltpu.sync_copy(accB.at[pl.ds(off, _W)], rB)
        pltpu.sync_copy(rB, outB.at[pl.ds(base + off, _W)])


def _mm_score_body(x_ref, w_ref, b_ref, ea_ref, nm_ref, s_ref):
    z = jnp.dot(x_ref[...], w_ref[...], preferred_element_type=jnp.float32)
    nm = jnp.maximum(z + b_ref[...], 0.0)
    nm_ref[...] = nm
    s_ref[...] = jnp.sum(nm * ea_ref[...], axis=1)[None, None, :]


_mm_score = pl.pallas_call(
    _mm_score_body,
    grid=(_NG,),
    in_specs=[
        pl.BlockSpec((_BLK, _D), lambda i: (i, 0)),
        pl.BlockSpec((_D, _D), lambda i: (0, 0)),
        pl.BlockSpec((1, _D), lambda i: (0, 0)),
        pl.BlockSpec((1, _D), lambda i: (0, 0)),
    ],
    out_specs=[
        pl.BlockSpec((_BLK, _D), lambda i: (i, 0)),
        pl.BlockSpec((1, 1, _BLK), lambda i: (i, 0, 0)),
    ],
    out_shape=[
        jax.ShapeDtypeStruct((_NP, _D), jnp.float32),
        jax.ShapeDtypeStruct((_NG, 1, _BLK), jnp.float32),
    ],
)


def _mm_relu_body(xa_ref, xb_ref, wa_ref, wb_ref, b_ref, oa_ref, ob_ref):
    z = jnp.dot(xa_ref[...], wa_ref[...], preferred_element_type=jnp.float32)
    z += jnp.dot(xb_ref[...], wb_ref[...], preferred_element_type=jnp.float32)
    z = jnp.maximum(z + b_ref[...], 0.0)
    oa_ref[...] = z[:, :_HD]
    ob_ref[...] = z[:, _HD:]


_mm_relu = pl.pallas_call(
    _mm_relu_body,
    grid=(_NG,),
    in_specs=[
        pl.BlockSpec((_BLK, _HD), lambda i: (i, 0)),
        pl.BlockSpec((_BLK, _HD), lambda i: (i, 0)),
        pl.BlockSpec((_HD, _D), lambda i: (0, 0)),
        pl.BlockSpec((_HD, _D), lambda i: (0, 0)),
        pl.BlockSpec((1, _D), lambda i: (0, 0)),
    ],
    out_specs=[
        pl.BlockSpec((_BLK, _HD), lambda i: (i, 0)),
        pl.BlockSpec((_BLK, _HD), lambda i: (i, 0)),
    ],
    out_shape=[
        jax.ShapeDtypeStruct((_NP, _HD), jnp.float32),
        jax.ShapeDtypeStruct((_NP, _HD), jnp.float32),
    ],
)


def _softmax_body(s_ref, c_ref, p_ref):
    s = s_ref[...]
    flat = (lax.broadcasted_iota(jnp.int32, s.shape, 0) * _BLK
            + lax.broadcasted_iota(jnp.int32, s.shape, 1))
    valid = flat < _N
    m = jnp.max(jnp.where(valid, s, -jnp.inf))
    texp = jnp.where(valid, jnp.exp(s - m), 0.0)
    z = jnp.sum(c_ref[...] * texp)
    p_ref[...] = texp / z


_softmax = pl.pallas_call(
    _softmax_body,
    out_shape=jax.ShapeDtypeStruct((_NG, _BLK), jnp.float32),
)


def _scale_body(nm_ref, p_ref, qa_ref, qb_ref):
    q = nm_ref[...] * p_ref[...]
    qa_ref[...] = q[:, :_HD]
    qb_ref[...] = q[:, _HD:]


_scale = pl.pallas_call(
    _scale_body,
    grid=(_NG,),
    in_specs=[
        pl.BlockSpec((_BLK, _D), lambda i: (i, 0)),
        pl.BlockSpec((_BLK, 1), lambda i: (i, 0)),
    ],
    out_specs=[
        pl.BlockSpec((_BLK, _HD), lambda i: (i, 0)),
        pl.BlockSpec((_BLK, _HD), lambda i: (i, 0)),
    ],
    out_shape=[
        jax.ShapeDtypeStruct((_NP, _HD), jnp.float32),
        jax.ShapeDtypeStruct((_NP, _HD), jnp.float32),
    ],
)


def _addscale_body(h_ref, ua_ref, ub_ref, p_ref, o_ref):
    u = jnp.concatenate([ua_ref[...], ub_ref[...]], axis=1)
    o_ref[...] = h_ref[...] + u * p_ref[...]


_addscale = pl.pallas_call(
    _addscale_body,
    grid=(_NG,),
    in_specs=[
        pl.BlockSpec((_BLK, _D), lambda i: (i, 0)),
        pl.BlockSpec((_BLK, _HD), lambda i: (i, 0)),
        pl.BlockSpec((_BLK, _HD), lambda i: (i, 0)),
        pl.BlockSpec((_BLK, 1), lambda i: (i, 0)),
    ],
    out_specs=pl.BlockSpec((_BLK, _D), lambda i: (i, 0)),
    out_shape=jax.ShapeDtypeStruct((_NP, _D), jnp.float32),
)


def kernel(node_feats, attr_feats, edge_index, Wn0, bn0, Wn1, bn1,
           Wa0, ba0, Wa1, ba1, edge_attention):
    src = edge_index[0]
    dst = edge_index[1]
    h = jnp.pad(node_feats, ((0, _NP - _N), (0, 0)))
    ea = edge_attention.reshape(1, _D)
    zeros2d = jnp.zeros((_W, _HD), jnp.float32)
    zeros1d = jnp.zeros((_NP // 16,), jnp.float32)

    c_src = _sc_hist(src, zeros1d).reshape(_NG, _BLK)

    haA = haB = None
    for (Wn, bn, Wa, ba) in ((Wn0, bn0, Wa0, ba0), (Wn1, bn1, Wa1, ba1)):
        nm, s = _mm_score(h, Wn, bn.reshape(1, _D), ea)
        p = _softmax(s.reshape(_NG, _BLK), c_src)
        pcol = p.reshape(_NP, 1)
        qA, qB = _scale(nm, pcol)
        amA, amB = _sc_pass(qA, qB, src, dst, zeros2d)
        haA, haB = _mm_relu(amA, amB, Wa[:_HD], Wa[_HD:], ba.reshape(1, _D))
        uA, uB = _sc_pass(haA, haB, dst, src, zeros2d)
        h = _addscale(h, uA, uB, pcol)
    ha = jnp.concatenate([haA, haB], axis=1)
    return h[:_N], ha[:_N]


# speedup vs baseline: 3.3584x; 1.4742x over previous
import functools

import jax
import jax.numpy as jnp
from jax import lax
from jax.experimental import pallas as pl
from jax.experimental.pallas import tpu as pltpu
from jax.experimental.pallas import tpu_sc as plsc

_N = 10000
_D = 256
_E = 160000
_NP = 10240
_HALF = _NP // 2
_STRIPE = _HALF // 16
_W = 80
_TPW = _E // 16
_NWIN = _TPW // _W
_BLK = 128
_NG = _NP // _BLK

_mesh = plsc.VectorSubcoreMesh(core_axis_name="c", subcore_axis_name="s")


@functools.partial(
    pl.kernel,
    out_type=jax.ShapeDtypeStruct((_NP,), jnp.float32),
    mesh=_mesh,
    scratch_types=[
        pltpu.VMEM_SHARED((_NP,), jnp.float32),
        pltpu.VMEM((_W,), jnp.int32),
        pltpu.VMEM((_W,), jnp.float32),
        pltpu.VMEM((_NP // 16,), jnp.float32),
        pltpu.SemaphoreType.DMA,
    ],
)
def _sc_hist(gidx, zeros1d, out, acc, idx_g, ones_v, zbuf, sem):
    c = lax.axis_index("c")
    t = lax.axis_index("s")
    seg = _NP // 16
    pltpu.sync_copy(zeros1d, zbuf)
    pltpu.sync_copy(zbuf, acc.at[pl.ds(t * seg, seg)])
    for j in range(_W // 16):
        ones_v[pl.ds(j * 16, 16)] = jnp.full((16,), 1.0, jnp.float32)
    plsc.subcore_barrier()

    ebase = t * _TPW

    def w_body(w, carry):
        eoff = pl.multiple_of(ebase + w * _W, 8)
        pltpu.sync_copy(gidx.at[pl.ds(eoff, _W)], idx_g)
        pltpu.sync_copy(ones_v, acc.at[idx_g], add=True)
        return carry

    lax.fori_loop(0, _NWIN, w_body, 0)
    plsc.subcore_barrier()
    off = c * _HALF + t * _STRIPE
    pltpu.sync_copy(acc.at[pl.ds(off, _STRIPE)], zbuf.at[pl.ds(0, _STRIPE)])
    pltpu.sync_copy(zbuf.at[pl.ds(0, _STRIPE)], out.at[pl.ds(off, _STRIPE)])


_HD = _D // 2


@functools.partial(
    pl.kernel,
    out_type=(jax.ShapeDtypeStruct((_NP, _HD), jnp.float32),
              jax.ShapeDtypeStruct((_NP, _HD), jnp.float32)),
    mesh=_mesh,
    scratch_types=[
        pltpu.VMEM_SHARED((_HALF + 8, _HD), jnp.float32),
        pltpu.VMEM_SHARED((_HALF + 8, _HD), jnp.float32),
        pltpu.VMEM((_W,), jnp.int32),
        pltpu.VMEM((_W,), jnp.int32),
        pltpu.VMEM((_W,), jnp.int32),
        pltpu.VMEM((_W,), jnp.int32),
        pltpu.VMEM((_W, _HD), jnp.float32),
        pltpu.VMEM((_W, _HD), jnp.float32),
        pltpu.VMEM((_W, _HD), jnp.float32),
        pltpu.VMEM((_W, _HD), jnp.float32),
        pltpu.SemaphoreType.DMA,
        pltpu.SemaphoreType.DMA,
    ],
)
def _sc_pass(tA, tB, gidx, sidx, zeros2d, outA, outB,
             accA, accB, idx_g0, idx_l0, idx_g1, idx_l1,
             rA0, rB0, rA1, rB1, sem0, sem1):
    c = lax.axis_index("c")
    t = lax.axis_index("s")
    base = c * _HALF
    pltpu.sync_copy(zeros2d, rA0)
    for k in range(_STRIPE // _W):
        pltpu.sync_copy(rA0, accA.at[pl.ds(t * _STRIPE + k * _W, _W)])
        pltpu.sync_copy(rA0, accB.at[pl.ds(t * _STRIPE + k * _W, _W)])

    plsc.subcore_barrier()

    ebase = t * _TPW

    def load_idx(w, ig, il):
        eoff = pl.multiple_of(jnp.minimum(ebase + w * _W, _E - _W), 8)
        pltpu.sync_copy(gidx.at[pl.ds(eoff, _W)], ig)
        pltpu.sync_copy(sidx.at[pl.ds(eoff, _W)], il)

    def mask_idx(ig, il):
        for j in range(_W // 16):
            d = il[pl.ds(j * 16, 16)]
            g = ig[pl.ds(j * 16, 16)]
            ld = d - base
            ok = (ld >= 0) & (ld < _HALF)
            il[pl.ds(j * 16, 16)] = jnp.where(ok, ld, -1)
            ig[pl.ds(j * 16, 16)] = jnp.where(ok, g, -1)

    def fire(ig, ra, rb, sem):
        gi = plsc.Indices(ig, ignored_value=-1)
        pltpu.async_copy(tA.at[gi], ra, sem)
        pltpu.async_copy(tB.at[gi], rb, sem)

    def drain(ig, ra, rb, sem):
        gi = plsc.Indices(ig, ignored_value=-1)
        pltpu.make_async_copy(tA.at[gi], ra, sem).wait()
        pltpu.make_async_copy(tB.at[gi], rb, sem).wait()

    def scatter(il, ra, rb):
        si = plsc.Indices(il, ignored_value=-1)
        pltpu.sync_copy(ra, accA.at[si], add=True)
        pltpu.sync_copy(rb, accB.at[si], add=True)

    load_idx(0, idx_g0, idx_l0)
    mask_idx(idx_g0, idx_l0)
    fire(idx_g0, rA0, rB0, sem0)
    load_idx(1, idx_g1, idx_l1)

    def w_body(k, carry):
        w0 = 2 * k
        mask_idx(idx_g1, idx_l1)
        fire(idx_g1, rA1, rB1, sem1)
        drain(idx_g0, rA0, rB0, sem0)
        scatter(idx_l0, rA0, rB0)
        load_idx(w0 + 2, idx_g0, idx_l0)
        mask_idx(idx_g0, idx_l0)
        fire(idx_g0, rA0, rB0, sem0)
        drain(idx_g1, rA1, rB1, sem1)
        scatter(idx_l1, rA1, rB1)
        load_idx(w0 + 3, idx_g1, idx_l1)
        return carry

    lax.fori_loop(0, (_NWIN - 1) // 2, w_body, 0)
    drain(idx_g0, rA0, rB0, sem0)
    scatter(idx_l0, rA0, rB0)

    plsc.subcore_barrier()
    for k in range(_STRIPE // _W):
        off = t * _STRIPE + k * _W
        pltpu.sync_copy(accA.at[pl.ds(off, _W)], rA0)
        pltpu.sync_copy(rA0, outA.at[pl.ds(base + off, _W)])
        pltpu.sync_copy(accB.at[pl.ds(off, _W)], rB0)
        pltpu.sync_copy(rB0, outB.at[pl.ds(base + off, _W)])


def _mm_score_body(x_ref, w_ref, b_ref, ea_ref, nm_ref, s_ref):
    z = jnp.dot(x_ref[...], w_ref[...], preferred_element_type=jnp.float32)
    nm = jnp.maximum(z + b_ref[...], 0.0)
    nm_ref[...] = nm
    s_ref[...] = jnp.sum(nm * ea_ref[...], axis=1)[None, None, :]


_mm_score = pl.pallas_call(
    _mm_score_body,
    grid=(_NG,),
    in_specs=[
        pl.BlockSpec((_BLK, _D), lambda i: (i, 0)),
        pl.BlockSpec((_D, _D), lambda i: (0, 0)),
        pl.BlockSpec((1, _D), lambda i: (0, 0)),
        pl.BlockSpec((1, _D), lambda i: (0, 0)),
    ],
    out_specs=[
        pl.BlockSpec((_BLK, _D), lambda i: (i, 0)),
        pl.BlockSpec((1, 1, _BLK), lambda i: (i, 0, 0)),
    ],
    out_shape=[
        jax.ShapeDtypeStruct((_NP, _D), jnp.float32),
        jax.ShapeDtypeStruct((_NG, 1, _BLK), jnp.float32),
    ],
)


def _mm_relu_body(xa_ref, xb_ref, wa_ref, wb_ref, b_ref, oa_ref, ob_ref):
    z = jnp.dot(xa_ref[...], wa_ref[...], preferred_element_type=jnp.float32)
    z += jnp.dot(xb_ref[...], wb_ref[...], preferred_element_type=jnp.float32)
    z = jnp.maximum(z + b_ref[...], 0.0)
    oa_ref[...] = z[:, :_HD]
    ob_ref[...] = z[:, _HD:]


_mm_relu = pl.pallas_call(
    _mm_relu_body,
    grid=(_NG,),
    in_specs=[
        pl.BlockSpec((_BLK, _HD), lambda i: (i, 0)),
        pl.BlockSpec((_BLK, _HD), lambda i: (i, 0)),
        pl.BlockSpec((_HD, _D), lambda i: (0, 0)),
        pl.BlockSpec((_HD, _D), lambda i: (0, 0)),
        pl.BlockSpec((1, _D), lambda i: (0, 0)),
    ],
    out_specs=[
        pl.BlockSpec((_BLK, _HD), lambda i: (i, 0)),
        pl.BlockSpec((_BLK, _HD), lambda i: (i, 0)),
    ],
    out_shape=[
        jax.ShapeDtypeStruct((_NP, _HD), jnp.float32),
        jax.ShapeDtypeStruct((_NP, _HD), jnp.float32),
    ],
)


def _softmax_body(s_ref, c_ref, p_ref):
    s = s_ref[...]
    flat = (lax.broadcasted_iota(jnp.int32, s.shape, 0) * _BLK
            + lax.broadcasted_iota(jnp.int32, s.shape, 1))
    valid = flat < _N
    m = jnp.max(jnp.where(valid, s, -jnp.inf))
    texp = jnp.where(valid, jnp.exp(s - m), 0.0)
    z = jnp.sum(c_ref[...] * texp)
    p_ref[...] = texp / z


_softmax = pl.pallas_call(
    _softmax_body,
    out_shape=jax.ShapeDtypeStruct((_NG, _BLK), jnp.float32),
)


def _scale_body(nm_ref, p_ref, qa_ref, qb_ref):
    q = nm_ref[...] * p_ref[...]
    qa_ref[...] = q[:, :_HD]
    qb_ref[...] = q[:, _HD:]


_scale = pl.pallas_call(
    _scale_body,
    grid=(_NG,),
    in_specs=[
        pl.BlockSpec((_BLK, _D), lambda i: (i, 0)),
        pl.BlockSpec((_BLK, 1), lambda i: (i, 0)),
    ],
    out_specs=[
        pl.BlockSpec((_BLK, _HD), lambda i: (i, 0)),
        pl.BlockSpec((_BLK, _HD), lambda i: (i, 0)),
    ],
    out_shape=[
        jax.ShapeDtypeStruct((_NP, _HD), jnp.float32),
        jax.ShapeDtypeStruct((_NP, _HD), jnp.float32),
    ],
)


def _addscale_body(h_ref, ua_ref, ub_ref, p_ref, o_ref):
    u = jnp.concatenate([ua_ref[...], ub_ref[...]], axis=1)
    o_ref[...] = h_ref[...] + u * p_ref[...]


_addscale = pl.pallas_call(
    _addscale_body,
    grid=(_NG,),
    in_specs=[
        pl.BlockSpec((_BLK, _D), lambda i: (i, 0)),
        pl.BlockSpec((_BLK, _HD), lambda i: (i, 0)),
        pl.BlockSpec((_BLK, _HD), lambda i: (i, 0)),
        pl.BlockSpec((_BLK, 1), lambda i: (i, 0)),
    ],
    out_specs=pl.BlockSpec((_BLK, _D), lambda i: (i, 0)),
    out_shape=jax.ShapeDtypeStruct((_NP, _D), jnp.float32),
)


def kernel(node_feats, attr_feats, edge_index, Wn0, bn0, Wn1, bn1,
           Wa0, ba0, Wa1, ba1, edge_attention):
    src = edge_index[0]
    dst = edge_index[1]
    h = jnp.pad(node_feats, ((0, _NP - _N), (0, 0)))
    ea = edge_attention.reshape(1, _D)
    zeros2d = jnp.zeros((_W, _HD), jnp.float32)
    zeros1d = jnp.zeros((_NP // 16,), jnp.float32)

    c_src = _sc_hist(src, zeros1d).reshape(_NG, _BLK)

    haA = haB = None
    for (Wn, bn, Wa, ba) in ((Wn0, bn0, Wa0, ba0), (Wn1, bn1, Wa1, ba1)):
        nm, s = _mm_score(h, Wn, bn.reshape(1, _D), ea)
        p = _softmax(s.reshape(_NG, _BLK), c_src)
        pcol = p.reshape(_NP, 1)
        qA, qB = _scale(nm, pcol)
        amA, amB = _sc_pass(qA, qB, src, dst, zeros2d)
        haA, haB = _mm_relu(amA, amB, Wa[:_HD], Wa[_HD:], ba.reshape(1, _D))
        uA, uB = _sc_pass(haA, haB, dst, src, zeros2d)
        h = _addscale(h, uA, uB, pcol)
    ha = jnp.concatenate([haA, haB], axis=1)
    return h[:_N], ha[:_N]


# speedup vs baseline: 4.7420x; 1.4120x over previous
import functools

import jax
import jax.numpy as jnp
from jax import lax
from jax.experimental import pallas as pl
from jax.experimental.pallas import tpu as pltpu
from jax.experimental.pallas import tpu_sc as plsc

_N = 10000
_D = 256
_E = 160000
_NP = 10240
_HALF = _NP // 2
_STRIPE = _HALF // 16
_W = 80
_TPW = _E // 16
_NWIN = _TPW // _W
_BLK = 512
_NG = _NP // _BLK

_mesh = plsc.VectorSubcoreMesh(core_axis_name="c", subcore_axis_name="s")


@functools.partial(
    pl.kernel,
    out_type=jax.ShapeDtypeStruct((_NP,), jnp.float32),
    mesh=_mesh,
    scratch_types=[
        pltpu.VMEM_SHARED((_NP,), jnp.float32),
        pltpu.VMEM((_W,), jnp.int32),
        pltpu.VMEM((_W,), jnp.float32),
        pltpu.VMEM((_NP // 16,), jnp.float32),
        pltpu.SemaphoreType.DMA,
    ],
)
def _sc_hist(gidx, zeros1d, out, acc, idx_g, ones_v, zbuf, sem):
    c = lax.axis_index("c")
    t = lax.axis_index("s")
    seg = _NP // 16
    pltpu.sync_copy(zeros1d, zbuf)
    pltpu.sync_copy(zbuf, acc.at[pl.ds(t * seg, seg)])
    for j in range(_W // 16):
        ones_v[pl.ds(j * 16, 16)] = jnp.full((16,), 1.0, jnp.float32)
    plsc.subcore_barrier()

    ebase = t * _TPW

    def w_body(w, carry):
        eoff = pl.multiple_of(ebase + w * _W, 8)
        pltpu.sync_copy(gidx.at[pl.ds(eoff, _W)], idx_g)
        pltpu.sync_copy(ones_v, acc.at[idx_g], add=True)
        return carry

    lax.fori_loop(0, _NWIN, w_body, 0)
    plsc.subcore_barrier()
    off = c * _HALF + t * _STRIPE
    pltpu.sync_copy(acc.at[pl.ds(off, _STRIPE)], zbuf.at[pl.ds(0, _STRIPE)])
    pltpu.sync_copy(zbuf.at[pl.ds(0, _STRIPE)], out.at[pl.ds(off, _STRIPE)])


_HD = _D // 2


@functools.partial(
    pl.kernel,
    out_type=(jax.ShapeDtypeStruct((_NP, _HD), jnp.float32),
              jax.ShapeDtypeStruct((_NP, _HD), jnp.float32)),
    mesh=_mesh,
    scratch_types=[
        pltpu.VMEM_SHARED((_HALF + 8, _HD), jnp.float32),
        pltpu.VMEM_SHARED((_HALF + 8, _HD), jnp.float32),
        pltpu.VMEM((_W,), jnp.int32),
        pltpu.VMEM((_W,), jnp.int32),
        pltpu.VMEM((_W,), jnp.int32),
        pltpu.VMEM((_W,), jnp.int32),
        pltpu.VMEM((_W, _HD), jnp.float32),
        pltpu.VMEM((_W, _HD), jnp.float32),
        pltpu.VMEM((_W, _HD), jnp.float32),
        pltpu.VMEM((_W, _HD), jnp.float32),
        pltpu.SemaphoreType.DMA,
        pltpu.SemaphoreType.DMA,
    ],
)
def _sc_pass(tA, tB, gidx, sidx, zeros2d, outA, outB,
             accA, accB, idx_g0, idx_l0, idx_g1, idx_l1,
             rA0, rB0, rA1, rB1, sem0, sem1):
    c = lax.axis_index("c")
    t = lax.axis_index("s")
    base = c * _HALF
    pltpu.sync_copy(zeros2d, rA0)
    for k in range(_STRIPE // _W):
        pltpu.sync_copy(rA0, accA.at[pl.ds(t * _STRIPE + k * _W, _W)])
        pltpu.sync_copy(rA0, accB.at[pl.ds(t * _STRIPE + k * _W, _W)])

    plsc.subcore_barrier()

    ebase = t * _TPW

    def load_idx(w, ig, il, sem):
        eoff = pl.multiple_of(jnp.minimum(ebase + w * _W, _E - _W), 8)
        pltpu.async_copy(gidx.at[pl.ds(eoff, _W)], ig, sem)
        cp = pltpu.async_copy(sidx.at[pl.ds(eoff, _W)], il, sem)
        pltpu.make_async_copy(gidx.at[pl.ds(eoff, _W)], ig, sem).wait()
        cp.wait()

    def mask_idx(ig, il):
        for j in range(_W // 16):
            d = il[pl.ds(j * 16, 16)]
            g = ig[pl.ds(j * 16, 16)]
            ld = d - base
            ok = (ld >= 0) & (ld < _HALF)
            il[pl.ds(j * 16, 16)] = jnp.where(ok, ld, -1)
            ig[pl.ds(j * 16, 16)] = jnp.where(ok, g, -1)

    def fire(ig, ra, rb, sem):
        gi = plsc.Indices(ig, ignored_value=-1)
        pltpu.async_copy(tA.at[gi], ra, sem)
        pltpu.async_copy(tB.at[gi], rb, sem)

    def drain(ig, ra, rb, sem):
        gi = plsc.Indices(ig, ignored_value=-1)
        pltpu.make_async_copy(tA.at[gi], ra, sem).wait()
        pltpu.make_async_copy(tB.at[gi], rb, sem).wait()

    def scatter(il, ra, rb, sem):
        si = plsc.Indices(il, ignored_value=-1)
        pltpu.async_copy(ra, accA.at[si], sem, add=True)
        cp = pltpu.async_copy(rb, accB.at[si], sem, add=True)
        pltpu.make_async_copy(ra, accA.at[si], sem).wait()
        cp.wait()

    load_idx(0, idx_g0, idx_l0, sem0)
    mask_idx(idx_g0, idx_l0)
    fire(idx_g0, rA0, rB0, sem0)
    load_idx(1, idx_g1, idx_l1, sem1)

    def w_body(k, carry):
        w0 = 2 * k
        mask_idx(idx_g1, idx_l1)
        fire(idx_g1, rA1, rB1, sem1)
        drain(idx_g0, rA0, rB0, sem0)
        scatter(idx_l0, rA0, rB0, sem0)
        load_idx(w0 + 2, idx_g0, idx_l0, sem0)
        mask_idx(idx_g0, idx_l0)
        fire(idx_g0, rA0, rB0, sem0)
        drain(idx_g1, rA1, rB1, sem1)
        scatter(idx_l1, rA1, rB1, sem1)
        load_idx(w0 + 3, idx_g1, idx_l1, sem1)
        return carry

    lax.fori_loop(0, (_NWIN - 1) // 2, w_body, 0)
    drain(idx_g0, rA0, rB0, sem0)
    scatter(idx_l0, rA0, rB0, sem0)

    plsc.subcore_barrier()
    for k in range(_STRIPE // _W):
        off = t * _STRIPE + k * _W
        pltpu.sync_copy(accA.at[pl.ds(off, _W)], rA0)
        pltpu.sync_copy(rA0, outA.at[pl.ds(base + off, _W)])
        pltpu.sync_copy(accB.at[pl.ds(off, _W)], rB0)
        pltpu.sync_copy(rB0, outB.at[pl.ds(base + off, _W)])


def _mm_score_body(x_ref, w_ref, b_ref, ea_ref, nm_ref, s_ref):
    z = jnp.dot(x_ref[...], w_ref[...], preferred_element_type=jnp.float32)
    nm = jnp.maximum(z + b_ref[...], 0.0)
    nm_ref[...] = nm
    s_ref[...] = jnp.sum(nm * ea_ref[...], axis=1)[None, None, :]


_mm_score = pl.pallas_call(
    _mm_score_body,
    grid=(_NG,),
    in_specs=[
        pl.BlockSpec((_BLK, _D), lambda i: (i, 0)),
        pl.BlockSpec((_D, _D), lambda i: (0, 0)),
        pl.BlockSpec((1, _D), lambda i: (0, 0)),
        pl.BlockSpec((1, _D), lambda i: (0, 0)),
    ],
    out_specs=[
        pl.BlockSpec((_BLK, _D), lambda i: (i, 0)),
        pl.BlockSpec((1, 1, _BLK), lambda i: (i, 0, 0)),
    ],
    out_shape=[
        jax.ShapeDtypeStruct((_NP, _D), jnp.float32),
        jax.ShapeDtypeStruct((_NG, 1, _BLK), jnp.float32),
    ],
)


def _mm_relu_body(xa_ref, xb_ref, wa_ref, wb_ref, b_ref, oa_ref, ob_ref):
    z = jnp.dot(xa_ref[...], wa_ref[...], preferred_element_type=jnp.float32)
    z += jnp.dot(xb_ref[...], wb_ref[...], preferred_element_type=jnp.float32)
    z = jnp.maximum(z + b_ref[...], 0.0)
    oa_ref[...] = z[:, :_HD]
    ob_ref[...] = z[:, _HD:]


_mm_relu = pl.pallas_call(
    _mm_relu_body,
    grid=(_NG,),
    in_specs=[
        pl.BlockSpec((_BLK, _HD), lambda i: (i, 0)),
        pl.BlockSpec((_BLK, _HD), lambda i: (i, 0)),
        pl.BlockSpec((_HD, _D), lambda i: (0, 0)),
        pl.BlockSpec((_HD, _D), lambda i: (0, 0)),
        pl.BlockSpec((1, _D), lambda i: (0, 0)),
    ],
    out_specs=[
        pl.BlockSpec((_BLK, _HD), lambda i: (i, 0)),
        pl.BlockSpec((_BLK, _HD), lambda i: (i, 0)),
    ],
    out_shape=[
        jax.ShapeDtypeStruct((_NP, _HD), jnp.float32),
        jax.ShapeDtypeStruct((_NP, _HD), jnp.float32),
    ],
)


def _softmax_body(s_ref, c_ref, p_ref):
    s = s_ref[...]
    flat = (lax.broadcasted_iota(jnp.int32, s.shape, 0) * _BLK
            + lax.broadcasted_iota(jnp.int32, s.shape, 1))
    valid = flat < _N
    m = jnp.max(jnp.where(valid, s, -jnp.inf))
    texp = jnp.where(valid, jnp.exp(s - m), 0.0)
    z = jnp.sum(c_ref[...] * texp)
    p_ref[...] = texp / z


_softmax = pl.pallas_call(
    _softmax_body,
    out_shape=jax.ShapeDtypeStruct((_NG, _BLK), jnp.float32),
)


def _scale_body(nm_ref, p_ref, qa_ref, qb_ref):
    q = nm_ref[...] * p_ref[...]
    qa_ref[...] = q[:, :_HD]
    qb_ref[...] = q[:, _HD:]


_scale = pl.pallas_call(
    _scale_body,
    grid=(_NG,),
    in_specs=[
        pl.BlockSpec((_BLK, _D), lambda i: (i, 0)),
        pl.BlockSpec((_BLK, 1), lambda i: (i, 0)),
    ],
    out_specs=[
        pl.BlockSpec((_BLK, _HD), lambda i: (i, 0)),
        pl.BlockSpec((_BLK, _HD), lambda i: (i, 0)),
    ],
    out_shape=[
        jax.ShapeDtypeStruct((_NP, _HD), jnp.float32),
        jax.ShapeDtypeStruct((_NP, _HD), jnp.float32),
    ],
)


def _addscale_body(h_ref, ua_ref, ub_ref, p_ref, o_ref):
    u = jnp.concatenate([ua_ref[...], ub_ref[...]], axis=1)
    o_ref[...] = h_ref[...] + u * p_ref[...]


_addscale = pl.pallas_call(
    _addscale_body,
    grid=(_NG,),
    in_specs=[
        pl.BlockSpec((_BLK, _D), lambda i: (i, 0)),
        pl.BlockSpec((_BLK, _HD), lambda i: (i, 0)),
        pl.BlockSpec((_BLK, _HD), lambda i: (i, 0)),
        pl.BlockSpec((_BLK, 1), lambda i: (i, 0)),
    ],
    out_specs=pl.BlockSpec((_BLK, _D), lambda i: (i, 0)),
    out_shape=jax.ShapeDtypeStruct((_NP, _D), jnp.float32),
)


def kernel(node_feats, attr_feats, edge_index, Wn0, bn0, Wn1, bn1,
           Wa0, ba0, Wa1, ba1, edge_attention):
    src = edge_index[0]
    dst = edge_index[1]
    h = jnp.pad(node_feats, ((0, _NP - _N), (0, 0)))
    ea = edge_attention.reshape(1, _D)
    zeros2d = jnp.zeros((_W, _HD), jnp.float32)
    zeros1d = jnp.zeros((_NP // 16,), jnp.float32)

    c_src = _sc_hist(src, zeros1d).reshape(_NG, _BLK)

    haA = haB = None
    for (Wn, bn, Wa, ba) in ((Wn0, bn0, Wa0, ba0), (Wn1, bn1, Wa1, ba1)):
        nm, s = _mm_score(h, Wn, bn.reshape(1, _D), ea)
        p = _softmax(s.reshape(_NG, _BLK), c_src)
        pcol = p.reshape(_NP, 1)
        qA, qB = _scale(nm, pcol)
        amA, amB = _sc_pass(qA, qB, src, dst, zeros2d)
        haA, haB = _mm_relu(amA, amB, Wa[:_HD], Wa[_HD:], ba.reshape(1, _D))
        uA, uB = _sc_pass(haA, haB, dst, src, zeros2d)
        h = _addscale(h, uA, uB, pcol)
    ha = jnp.concatenate([haA, haB], axis=1)
    return h[:_N], ha[:_N]
